# baseline (speedup 1.0000x reference)
import functools
import jax
import jax.numpy as jnp
from jax import lax
from jax.experimental import pallas as pl
from jax.experimental.pallas import tpu as pltpu
from jax.experimental.pallas import tpu_sc as plsc

N_NODES_K = 10000
N_EDGES_K = 320000
N_ANGLES_K = 480000
NUM_GRAPHS_K = 512
HID_K = 64
ATT_HEADS_K = 4
T_K = NUM_GRAPHS_K

_NC, _NS = 2, 16
_NW = _NC * _NS

E_PAD = 327680
A_PAD = 491520
NPOOL = 32768
NPAD_NODE = 10112
HALF_NODE = 5000
NPH_NODE = 5120
HALF_POOL = 256
NPH_POOL = 512


_CH = 512


def _gather_pair(table, idx_l, idx_r, B):
    per_w = B // _NW
    nch = per_w // _CH
    mesh = plsc.VectorSubcoreMesh(core_axis_name="c", subcore_axis_name="s")

    @functools.partial(
        pl.kernel, mesh=mesh,
        out_type=(jax.ShapeDtypeStruct((B, 128), jnp.float32),
                  jax.ShapeDtypeStruct((B, 128), jnp.float32)),
        scratch_types=[
            pltpu.VMEM((_CH,), jnp.int32),
            pltpu.VMEM((_CH,), jnp.int32),
            pltpu.VMEM((_CH, 128), jnp.float32),
            pltpu.SemaphoreType.DMA,
        ])
    def k(t_hbm, il_hbm, ir_hbm, gl_hbm, gr_hbm, il_v, ir_v, rows_v, sem):
        wid = lax.axis_index("s") * _NC + lax.axis_index("c")
        base = wid * per_w

        def body(i, carry):
            off = base + i * _CH
            pltpu.sync_copy(il_hbm.at[pl.ds(off, _CH)], il_v)
            pltpu.sync_copy(ir_hbm.at[pl.ds(off, _CH)], ir_v)
            pltpu.async_copy(t_hbm.at[il_v], rows_v, sem).wait()
            pltpu.sync_copy(rows_v, gl_hbm.at[pl.ds(off, _CH)])
            pltpu.async_copy(t_hbm.at[ir_v], rows_v, sem).wait()
            pltpu.sync_copy(rows_v, gr_hbm.at[pl.ds(off, _CH)])
            return carry

        lax.fori_loop(0, nch, body, 0)

    return k(table, idx_l, idx_r)


def _sc_scatter_add(w, dst_split, n_ph, B):
    per_t = B // 16
    nch = per_t // _CH
    rows_pt = n_ph // 16
    zeros = jnp.zeros((n_ph, 128), jnp.float32)
    mesh = plsc.VectorSubcoreMesh(core_axis_name="c", subcore_axis_name="s")

    @functools.partial(
        pl.kernel, mesh=mesh,
        out_type=jax.ShapeDtypeStruct((_NC, n_ph, 128), jnp.float32),
        scratch_types=[
            pltpu.VMEM((_CH,), jnp.int32),
            pltpu.VMEM((_CH, 128), jnp.float32),
            pltpu.VMEM_SHARED((n_ph, 128), jnp.float32),
        ])
    def k(w_hbm, dst_hbm, z_hbm, out_hbm, idx_v, rows_v, acc):
        cid = lax.axis_index("c")
        sid = lax.axis_index("s")
        r0 = sid * rows_pt
        pltpu.sync_copy(z_hbm.at[pl.ds(r0, rows_pt)], acc.at[pl.ds(r0, rows_pt)])
        plsc.subcore_barrier()
        base = sid * per_t

        def body(i, carry):
            off = base + i * _CH
            pltpu.sync_copy(dst_hbm.at[cid, pl.ds(off, _CH)], idx_v)
            pltpu.sync_copy(w_hbm.at[pl.ds(off, _CH)], rows_v)
            pltpu.sync_copy(rows_v, acc.at[idx_v], add=True)
            return carry

        lax.fori_loop(0, nch, body, 0)
        plsc.subcore_barrier()
        pltpu.sync_copy(acc.at[pl.ds(r0, rows_pt)],
                        out_hbm.at[cid, pl.ds(r0, rows_pt)])

    return k(w, dst_split, zeros)


def _split_idx(idx_pad, half, trash):
    lo = jnp.where(idx_pad < half, idx_pad, trash)
    hi_ok = (idx_pad >= half) & (idx_pad < 2 * half)
    hi = jnp.where(hi_ok, idx_pad - half, trash)
    return jnp.stack([lo, hi])


def _proj2_kernel(x_ref, wl_ref, bl_ref, wr_ref, br_ref, o_ref):
    xv = x_ref[...]
    xl = jnp.dot(xv, wl_ref[...], preferred_element_type=jnp.float32)
    xr = jnp.dot(xv, wr_ref[...], preferred_element_type=jnp.float32)
    o_ref[...] = jnp.concatenate([xl + bl_ref[...], xr + br_ref[...]], axis=1)


def _proj2_small(x, p):
    n = x.shape[0]
    return pl.pallas_call(
        _proj2_kernel,
        out_shape=jax.ShapeDtypeStruct((n, 128), jnp.float32),
    )(x, p['Wl'], p['bl'][None, :], p['Wr'], p['br'][None, :])


def _proj2_blocked(x, p, BLK=4096):
    n, de = x.shape
    return pl.pallas_call(
        _proj2_kernel,
        grid=(n // BLK,),
        in_specs=[pl.BlockSpec((BLK, de), lambda i: (i, 0)),
                  pl.BlockSpec((de, HID_K), lambda i: (0, 0)),
                  pl.BlockSpec((1, HID_K), lambda i: (0, 0)),
                  pl.BlockSpec((de, HID_K), lambda i: (0, 0)),
                  pl.BlockSpec((1, HID_K), lambda i: (0, 0))],
        out_specs=pl.BlockSpec((BLK, 128), lambda i: (i, 0)),
        out_shape=jax.ShapeDtypeStruct((n, 128), jnp.float32),
    )(x, p['Wl'], p['bl'][None, :], p['Wr'], p['br'][None, :])


def _edge_w_kernel(gl_ref, gr_ref, e_ref, we_ref, att_ref, w_ref):
    xlgv = gl_ref[:, 0:HID_K]
    xrgv = gr_ref[:, HID_K:128]
    m = xlgv + xrgv + jnp.dot(e_ref[...], we_ref[...],
                              preferred_element_type=jnp.float32)
    m = jnp.where(m > 0, m, 0.2 * m)
    ex = jnp.exp(jnp.dot(m, att_ref[...],
                         preferred_element_type=jnp.float32))
    blk = xlgv.shape[0]
    w_ref[...] = jnp.concatenate(
        [ex * xlgv, ex, jnp.zeros((blk, 127 - HID_K), jnp.float32)], axis=1)


def _edge_w(gl, gr, e, p, B, BLK=2048):
    de = e.shape[1]
    return pl.pallas_call(
        _edge_w_kernel,
        grid=(B // BLK,),
        in_specs=[pl.BlockSpec((BLK, 128), lambda i: (i, 0)),
                  pl.BlockSpec((BLK, 128), lambda i: (i, 0)),
                  pl.BlockSpec((BLK, de), lambda i: (i, 0)),
                  pl.BlockSpec((de, HID_K), lambda i: (0, 0)),
                  pl.BlockSpec((HID_K, 1), lambda i: (0, 0))],
        out_specs=pl.BlockSpec((BLK, 128), lambda i: (i, 0)),
        out_shape=jax.ShapeDtypeStruct((B, 128), jnp.float32),
    )(gl, gr, e, p['We'], p['att'][:, None])


def _finalize_proj_kernel(p_ref, bprev_ref, wl_ref, bl_ref, wr_ref, br_ref,
                          o_ref):
    acc = jnp.concatenate([p_ref[0, :HALF_NODE], p_ref[1, :HALF_NODE]], axis=0)
    num = acc[:, 0:HID_K]
    den = acc[:, HID_K:HID_K + 1]
    h = jnp.maximum(num / (den + 1e-16) + bprev_ref[...], 0.0)
    xl = jnp.dot(h, wl_ref[...], preferred_element_type=jnp.float32)
    xr = jnp.dot(h, wr_ref[...], preferred_element_type=jnp.float32)
    o_ref[...] = jnp.concatenate([xl + bl_ref[...], xr + br_ref[...]], axis=1)


def _finalize_proj(parts, b_prev, p):
    return pl.pallas_call(
        _finalize_proj_kernel,
        out_shape=jax.ShapeDtypeStruct((N_NODES_K, 128), jnp.float32),
    )(parts, b_prev[None, :], p['Wl'], p['bl'][None, :],
      p['Wr'], p['br'][None, :])


def _finalize_pool_kernel(p_ref, bprev_ref, out_ref):
    acc = jnp.concatenate([p_ref[0, :HALF_NODE], p_ref[1, :HALF_NODE]], axis=0)
    num = acc[:, 0:HID_K]
    den = acc[:, HID_K:HID_K + 1]
    h = jnp.maximum(num / (den + 1e-16) + bprev_ref[...], 0.0)
    pad = NPAD_NODE - N_NODES_K
    rows = jnp.concatenate(
        [h, jnp.ones((N_NODES_K, 1), jnp.float32),
         jnp.zeros((N_NODES_K, 127 - HID_K), jnp.float32)], axis=1)
    out_ref[...] = jnp.concatenate(
        [rows, jnp.zeros((pad, 128), jnp.float32)], axis=0)


def _finalize_pool(parts, b_prev):
    return pl.pallas_call(
        _finalize_pool_kernel,
        out_shape=jax.ShapeDtypeStruct((NPAD_NODE, 128), jnp.float32),
    )(parts, b_prev[None, :])


def _lstm_proj_kernel(x0_ref, x1_ref, x2_ref, w0_ref, w1_ref, w2_ref,
                      b_ref, out_ref):
    ps = []
    for x_ref, w_ref in ((x0_ref, w0_ref), (x1_ref, w1_ref), (x2_ref, w2_ref)):
        ps.append(jax.lax.dot_general(
            x_ref[...], w_ref[...], (((1,), (0,)), ((), ())),
            preferred_element_type=jnp.float32))
    for l in range(3):
        for g in range(4):
            out_ref[:, g * 192 + l * 64:g * 192 + (l + 1) * 64] = (
                ps[l][:, g * 64:(g + 1) * 64]
                + b_ref[0, g * 192 + l * 64:g * 192 + (l + 1) * 64])


def _lstm_rec_kernel(gin_ref, whh_ref, out_ref):
    def step(t, carry):
        h, c = carry
        g = gin_ref[t, :][None, :] + jax.lax.dot_general(
            h, whh_ref[...], (((1,), (0,)), ((), ())),
            preferred_element_type=jnp.float32)
        i = jax.nn.sigmoid(g[:, 0:192])
        f = jax.nn.sigmoid(g[:, 192:384])
        gg = jnp.tanh(g[:, 384:576])
        o = jax.nn.sigmoid(g[:, 576:768])
        c2 = f * c + i * gg
        h2 = o * jnp.tanh(c2)
        out_ref[t, :] = h2[0]
        return (h2, c2)
    h0 = jnp.zeros((1, 192), jnp.float32)
    jax.lax.fori_loop(0, T_K, step, (h0, h0))


def _lstm_block(sub_f, pub_f, maccs_f, p0, p1, p2):
    ws = [p['Wih'].T for p in (p0, p1, p2)]
    bs = [p['bih'] + p['bhh'] for p in (p0, p1, p2)]
    bias = jnp.zeros((1, 768), jnp.float32)
    whh_bd = jnp.zeros((192, 768), jnp.float32)
    for l, p in enumerate((p0, p1, p2)):
        wt = p['Whh'].T
        for g in range(4):
            whh_bd = whh_bd.at[l * 64:(l + 1) * 64,
                               g * 192 + l * 64:g * 192 + (l + 1) * 64].set(
                                   wt[:, g * 64:(g + 1) * 64])
            bias = bias.at[0, g * 192 + l * 64:g * 192 + (l + 1) * 64].set(
                bs[l][g * 64:(g + 1) * 64])
    gin = pl.pallas_call(
        _lstm_proj_kernel,
        out_shape=jax.ShapeDtypeStruct((T_K, 768), jnp.float32),
    )(sub_f, pub_f, maccs_f, ws[0], ws[1], ws[2], bias)
    hs = pl.pallas_call(
        _lstm_rec_kernel,
        out_shape=jax.ShapeDtypeStruct((T_K, 192), jnp.float32),
    )(gin, whh_bd)
    return hs[:, 0:64], hs[:, 64:128], hs[:, 128:192]


def _bn_eval(x, p):
    return x / jnp.sqrt(1.0 + 1e-5) * p['gamma'] + p['beta']


def _fusion_kernel(fp_ref, pp_ref, gam_ref, bet_ref, wfp_ref, bfp_ref,
                   qr_ref, wg_ref, bg_ref, qs_ref, wf_ref, bf_ref,
                   ow_ref, ob_ref, out_ref):
    fp = fp_ref[...]
    acc = jnp.concatenate([pp_ref[0, :HALF_POOL], pp_ref[1, :HALF_POOL]],
                          axis=0)
    num = acc[:, 0:HID_K]
    cnt = acc[:, HID_K:HID_K + 1]
    g0 = num / jnp.maximum(cnt, 1.0)
    g = g0 / jnp.sqrt(1.0 + 1e-5) * gam_ref[...] + bet_ref[...]
    fpf = jnp.tanh(fp @ wfp_ref[...] + bfp_ref[...])
    grf = jnp.tanh(g @ wg_ref[...] + bg_ref[...])
    fa = jnp.exp(fpf @ qr_ref[...])
    ga = jnp.exp(grf @ qs_ref[...])
    den2 = fa + ga
    fa = fa / den2
    ga = ga / den2
    fx = jnp.concatenate(
        [fa[:, i:i + 1] * fpf + ga[:, i:i + 1] * grf
         for i in range(ATT_HEADS_K)], axis=1)
    fx = fx @ wf_ref[...] + bf_ref[...]
    out_ref[...] = fx @ ow_ref[...] + ob_ref[...]


def _pad1(a, n_pad, fill):
    return jnp.concatenate(
        [a.astype(jnp.int32), jnp.full((n_pad - a.shape[0],), fill, jnp.int32)])


def kernel(x, edge_attr, angle_attr, sub_f, pub_f, maccs_f, edge_index, angle_index, batch, params):
    src, dst = edge_index[0], edge_index[1]
    asrc, adst = angle_index[0], angle_index[1]

    src2 = _pad1(src, E_PAD, 0)
    dstg2 = _pad1(dst, E_PAD, 0)
    dsts2 = _split_idx(_pad1(dst, E_PAD, N_NODES_K), HALF_NODE, 5056)
    asrc2 = _pad1(asrc, A_PAD, 0)
    adstg2 = _pad1(adst, A_PAD, 0)
    ea_pad = jnp.zeros((E_PAD, 16), jnp.float32).at[:N_EDGES_K].set(edge_attr)
    aa_pad = jnp.zeros((A_PAD, 4), jnp.float32).at[:N_ANGLES_K].set(angle_attr)

    def node_conv(xlr, e_pad, p):
        gl, gr = _gather_pair(xlr, src2, dstg2, E_PAD)
        w = _edge_w(gl, gr, e_pad, p, E_PAD)
        return _sc_scatter_add(w, dsts2, NPH_NODE, E_PAD)

    def angle_conv(x_e_pad, p):
        xlr = _proj2_blocked(x_e_pad, p)
        gl, gr = _gather_pair(xlr, asrc2, adstg2, A_PAD)
        w = _edge_w(gl, gr, aa_pad, p, A_PAD)
        acc = jax.ops.segment_sum(w[:N_ANGLES_K, :HID_K + 1], adst,
                                  num_segments=N_EDGES_K)
        ba = acc[:, :HID_K] / (acc[:, HID_K:HID_K + 1] + 1e-16) + p['b']
        return jnp.zeros((E_PAD, HID_K), jnp.float32).at[:N_EDGES_K].set(ba)

    xlr1 = _proj2_small(x, params['conv1'])
    parts1 = node_conv(xlr1, ea_pad, params['conv1'])
    ba_pad = angle_conv(ea_pad, params['hconv1'])

    xlr2 = _finalize_proj(parts1, params['conv1']['b'], params['conv2'])
    parts2 = node_conv(xlr2, ba_pad, params['conv2'])
    ba_pad = angle_conv(ba_pad, params['hconv2'])

    xlr3 = _finalize_proj(parts2, params['conv2']['b'], params['conv3'])
    parts3 = node_conv(xlr3, ba_pad, params['conv3'])

    hrows = _finalize_pool(parts3, params['conv3']['b'])
    hrows_p = jnp.zeros((NPOOL, 128), jnp.float32).at[:NPAD_NODE].set(hrows)
    batch2 = _split_idx(_pad1(batch, NPOOL, NUM_GRAPHS_K), HALF_POOL,
                        HALF_POOL)
    pool_parts = _sc_scatter_add(hrows_p, batch2, NPH_POOL, NPOOL)

    s, pu, mc = _lstm_block(sub_f, pub_f, maccs_f,
                            params['lstm0'], params['lstm1'], params['lstm2'])
    fus = jnp.stack([s, pu, mc], axis=1)
    y = jnp.mean(fus, axis=(1, 2))
    yp = jnp.pad(y, 1)
    w = params['eca_w']
    cw = jax.nn.sigmoid(w[0] * yp[:-2] + w[1] * yp[1:-1] + w[2] * yp[2:])
    fp = cw[:, None] * jnp.sum(fus, axis=1)
    fp = _bn_eval(fp, params['f_bn'])

    a = params['att']
    out = pl.pallas_call(
        _fusion_kernel,
        out_shape=jax.ShapeDtypeStruct((NUM_GRAPHS_K, 17), jnp.float32),
    )(fp, pool_parts, params['g_bn']['gamma'][None, :],
      params['g_bn']['beta'][None, :],
      a['Wfp'], a['bfp'], a['qr'], a['Wg'], a['bg'], a['qs'],
      a['Wf'], a['bf'], params['out_W'], params['out_b'])
    return out


# speedup vs baseline: 1.0537x; 1.0537x over previous
import functools
import jax
import jax.numpy as jnp
from jax import lax
from jax.experimental import pallas as pl
from jax.experimental.pallas import tpu as pltpu
from jax.experimental.pallas import tpu_sc as plsc

N_NODES_K = 10000
N_EDGES_K = 320000
N_ANGLES_K = 480000
NUM_GRAPHS_K = 512
HID_K = 64
ATT_HEADS_K = 4
T_K = NUM_GRAPHS_K

_NC, _NS = 2, 16
_NW = _NC * _NS

E_PAD = 327680
A_PAD = 491520
NPOOL = 32768
NPAD_NODE = 10112
HALF_NODE = 5000
NPH_NODE = 5120
HALF_POOL = 256
NPH_POOL = 512


_CH = 512


def _gather_pair(table, idx_l, idx_r, B):
    per_w = B // _NW
    CH = 160
    nch = per_w // CH
    mesh = plsc.VectorSubcoreMesh(core_axis_name="c", subcore_axis_name="s")

    @functools.partial(
        pl.kernel, mesh=mesh,
        out_type=(jax.ShapeDtypeStruct((B, 128), jnp.float32),
                  jax.ShapeDtypeStruct((B, 128), jnp.float32)),
        scratch_types=[
            pltpu.VMEM((per_w,), jnp.int32),
            pltpu.VMEM((per_w,), jnp.int32),
            pltpu.VMEM((CH, 128), jnp.float32),
            pltpu.VMEM((CH, 128), jnp.float32),
            pltpu.VMEM((CH, 128), jnp.float32),
            pltpu.VMEM((CH, 128), jnp.float32),
            pltpu.SemaphoreType.DMA,
            pltpu.SemaphoreType.DMA,
            pltpu.SemaphoreType.DMA,
            pltpu.SemaphoreType.DMA,
        ])
    def k(t_hbm, il_hbm, ir_hbm, gl_hbm, gr_hbm,
          il_all, ir_all, rl0, rl1, rr0, rr1, sl0, sl1, sr0, sr1):
        wid = lax.axis_index("s") * _NC + lax.axis_index("c")
        base = wid * per_w
        pltpu.sync_copy(il_hbm.at[pl.ds(base, per_w)], il_all)
        pltpu.sync_copy(ir_hbm.at[pl.ds(base, per_w)], ir_all)
        rl, rr = (rl0, rl1), (rr0, rr1)
        sl, sr = (sl0, sl1), (sr0, sr1)

        def fire(j, b):
            pltpu.async_copy(t_hbm.at[il_all.at[pl.ds(j * CH, CH)]],
                             rl[b], sl[b])
            pltpu.async_copy(t_hbm.at[ir_all.at[pl.ds(j * CH, CH)]],
                             rr[b], sr[b])

        def waitg(j, b):
            pltpu.make_async_copy(t_hbm.at[il_all.at[pl.ds(j * CH, CH)]],
                                  rl[b], sl[b]).wait()
            pltpu.make_async_copy(t_hbm.at[ir_all.at[pl.ds(j * CH, CH)]],
                                  rr[b], sr[b]).wait()

        for b in range(2):
            fire(b, b)

        def outer(io, carry):
            for b in range(2):
                i = io * 2 + b
                waitg(i, b)
                off = base + i * CH
                pltpu.sync_copy(rl[b], gl_hbm.at[pl.ds(off, CH)])
                pltpu.sync_copy(rr[b], gr_hbm.at[pl.ds(off, CH)])
                nxt = i + 2

                @pl.when(nxt < nch)
                def _():
                    fire(nxt, b)
            return carry

        lax.fori_loop(0, nch // 2, outer, 0)

    return k(table, idx_l, idx_r)


def _sc_scatter_add(w, dst_split, n_ph, B):
    per_t = B // 16
    CH = 320 if per_t % 640 == 0 else 256
    nch = per_t // CH
    rows_pt = n_ph // 16
    zeros = jnp.zeros((n_ph, 128), jnp.float32)
    mesh = plsc.VectorSubcoreMesh(core_axis_name="c", subcore_axis_name="s")

    @functools.partial(
        pl.kernel, mesh=mesh,
        out_type=jax.ShapeDtypeStruct((_NC, n_ph, 128), jnp.float32),
        scratch_types=[
            pltpu.VMEM((CH,), jnp.int32),
            pltpu.VMEM((CH,), jnp.int32),
            pltpu.VMEM((CH, 128), jnp.float32),
            pltpu.VMEM((CH, 128), jnp.float32),
            pltpu.SemaphoreType.DMA,
            pltpu.SemaphoreType.DMA,
            pltpu.SemaphoreType.DMA,
            pltpu.SemaphoreType.DMA,
            pltpu.VMEM_SHARED((n_ph, 128), jnp.float32),
        ])
    def k(w_hbm, dst_hbm, z_hbm, out_hbm,
          i0, i1, r0b, r1b, si0, si1, sr0, sr1, acc):
        cid = lax.axis_index("c")
        sid = lax.axis_index("s")
        r0 = sid * rows_pt
        pltpu.sync_copy(z_hbm.at[pl.ds(r0, rows_pt)], acc.at[pl.ds(r0, rows_pt)])
        plsc.subcore_barrier()
        base = sid * per_t
        iv, rv = (i0, i1), (r0b, r1b)
        si, sr = (si0, si1), (sr0, sr1)

        def fire(j, b):
            off = base + j * CH
            pltpu.async_copy(dst_hbm.at[pl.ds(cid * B + off, CH)], iv[b], si[b])
            pltpu.async_copy(w_hbm.at[pl.ds(off, CH)], rv[b], sr[b])

        def waitf(j, b):
            off = base + j * CH
            pltpu.make_async_copy(dst_hbm.at[pl.ds(cid * B + off, CH)],
                                  iv[b], si[b]).wait()
            pltpu.make_async_copy(w_hbm.at[pl.ds(off, CH)],
                                  rv[b], sr[b]).wait()

        for b in range(2):
            fire(b, b)

        def outer(io, carry):
            for b in range(2):
                i = io * 2 + b
                waitf(i, b)
                pltpu.sync_copy(rv[b], acc.at[iv[b]], add=True)
                nxt = i + 2

                @pl.when(nxt < nch)
                def _():
                    fire(nxt, b)
            return carry

        lax.fori_loop(0, nch // 2, outer, 0)
        plsc.subcore_barrier()
        pltpu.sync_copy(acc.at[pl.ds(r0, rows_pt)],
                        out_hbm.at[cid, pl.ds(r0, rows_pt)])

    return k(w, dst_split, zeros)


def _split_idx(idx_pad, half, trash):
    lo = jnp.where(idx_pad < half, idx_pad, trash)
    hi_ok = (idx_pad >= half) & (idx_pad < 2 * half)
    hi = jnp.where(hi_ok, idx_pad - half, trash)
    return jnp.concatenate([lo, hi])


def _proj2_kernel(x_ref, wl_ref, bl_ref, wr_ref, br_ref, o_ref):
    xv = x_ref[...]
    xl = jnp.dot(xv, wl_ref[...], preferred_element_type=jnp.float32)
    xr = jnp.dot(xv, wr_ref[...], preferred_element_type=jnp.float32)
    o_ref[...] = jnp.concatenate([xl + bl_ref[...], xr + br_ref[...]], axis=1)


def _proj2_small(x, p):
    n = x.shape[0]
    return pl.pallas_call(
        _proj2_kernel,
        out_shape=jax.ShapeDtypeStruct((n, 128), jnp.float32),
    )(x, p['Wl'], p['bl'][None, :], p['Wr'], p['br'][None, :])


def _proj2_blocked(x, p, BLK=4096):
    n, de = x.shape
    return pl.pallas_call(
        _proj2_kernel,
        grid=(n // BLK,),
        in_specs=[pl.BlockSpec((BLK, de), lambda i: (i, 0)),
                  pl.BlockSpec((de, HID_K), lambda i: (0, 0)),
                  pl.BlockSpec((1, HID_K), lambda i: (0, 0)),
                  pl.BlockSpec((de, HID_K), lambda i: (0, 0)),
                  pl.BlockSpec((1, HID_K), lambda i: (0, 0))],
        out_specs=pl.BlockSpec((BLK, 128), lambda i: (i, 0)),
        out_shape=jax.ShapeDtypeStruct((n, 128), jnp.float32),
    )(x, p['Wl'], p['bl'][None, :], p['Wr'], p['br'][None, :])


def _edge_w_kernel(gl_ref, gr_ref, e_ref, we_ref, att_ref, w_ref):
    xlgv = gl_ref[:, 0:HID_K]
    xrgv = gr_ref[:, HID_K:128]
    m = xlgv + xrgv + jnp.dot(e_ref[...], we_ref[...],
                              preferred_element_type=jnp.float32)
    m = jnp.where(m > 0, m, 0.2 * m)
    ex = jnp.exp(jnp.dot(m, att_ref[...],
                         preferred_element_type=jnp.float32))
    blk = xlgv.shape[0]
    w_ref[...] = jnp.concatenate(
        [ex * xlgv, ex, jnp.zeros((blk, 127 - HID_K), jnp.float32)], axis=1)


def _edge_w(gl, gr, e, p, B, BLK=2048):
    de = e.shape[1]
    return pl.pallas_call(
        _edge_w_kernel,
        grid=(B // BLK,),
        in_specs=[pl.BlockSpec((BLK, 128), lambda i: (i, 0)),
                  pl.BlockSpec((BLK, 128), lambda i: (i, 0)),
                  pl.BlockSpec((BLK, de), lambda i: (i, 0)),
                  pl.BlockSpec((de, HID_K), lambda i: (0, 0)),
                  pl.BlockSpec((HID_K, 1), lambda i: (0, 0))],
        out_specs=pl.BlockSpec((BLK, 128), lambda i: (i, 0)),
        out_shape=jax.ShapeDtypeStruct((B, 128), jnp.float32),
    )(gl, gr, e, p['We'], p['att'][:, None])


def _finalize_proj_kernel(p_ref, bprev_ref, wl_ref, bl_ref, wr_ref, br_ref,
                          o_ref):
    acc = jnp.concatenate([p_ref[0, :HALF_NODE], p_ref[1, :HALF_NODE]], axis=0)
    num = acc[:, 0:HID_K]
    den = acc[:, HID_K:HID_K + 1]
    h = jnp.maximum(num / (den + 1e-16) + bprev_ref[...], 0.0)
    xl = jnp.dot(h, wl_ref[...], preferred_element_type=jnp.float32)
    xr = jnp.dot(h, wr_ref[...], preferred_element_type=jnp.float32)
    o_ref[...] = jnp.concatenate([xl + bl_ref[...], xr + br_ref[...]], axis=1)


def _finalize_proj(parts, b_prev, p):
    return pl.pallas_call(
        _finalize_proj_kernel,
        out_shape=jax.ShapeDtypeStruct((N_NODES_K, 128), jnp.float32),
    )(parts, b_prev[None, :], p['Wl'], p['bl'][None, :],
      p['Wr'], p['br'][None, :])


def _finalize_pool_kernel(p_ref, bprev_ref, out_ref):
    acc = jnp.concatenate([p_ref[0, :HALF_NODE], p_ref[1, :HALF_NODE]], axis=0)
    num = acc[:, 0:HID_K]
    den = acc[:, HID_K:HID_K + 1]
    h = jnp.maximum(num / (den + 1e-16) + bprev_ref[...], 0.0)
    pad = NPAD_NODE - N_NODES_K
    rows = jnp.concatenate(
        [h, jnp.ones((N_NODES_K, 1), jnp.float32),
         jnp.zeros((N_NODES_K, 127 - HID_K), jnp.float32)], axis=1)
    out_ref[...] = jnp.concatenate(
        [rows, jnp.zeros((pad, 128), jnp.float32)], axis=0)


def _finalize_pool(parts, b_prev):
    return pl.pallas_call(
        _finalize_pool_kernel,
        out_shape=jax.ShapeDtypeStruct((NPAD_NODE, 128), jnp.float32),
    )(parts, b_prev[None, :])


def _lstm_proj_kernel(x0_ref, x1_ref, x2_ref, w0_ref, w1_ref, w2_ref,
                      b_ref, out_ref):
    ps = []
    for x_ref, w_ref in ((x0_ref, w0_ref), (x1_ref, w1_ref), (x2_ref, w2_ref)):
        ps.append(jax.lax.dot_general(
            x_ref[...], w_ref[...], (((1,), (0,)), ((), ())),
            preferred_element_type=jnp.float32))
    for l in range(3):
        for g in range(4):
            out_ref[:, g * 192 + l * 64:g * 192 + (l + 1) * 64] = (
                ps[l][:, g * 64:(g + 1) * 64]
                + b_ref[0, g * 192 + l * 64:g * 192 + (l + 1) * 64])


def _lstm_rec_kernel(gin_ref, whh_ref, out_ref):
    def step(t, carry):
        h, c = carry
        g = gin_ref[t, :][None, :] + jax.lax.dot_general(
            h, whh_ref[...], (((1,), (0,)), ((), ())),
            preferred_element_type=jnp.float32)
        i = jax.nn.sigmoid(g[:, 0:192])
        f = jax.nn.sigmoid(g[:, 192:384])
        gg = jnp.tanh(g[:, 384:576])
        o = jax.nn.sigmoid(g[:, 576:768])
        c2 = f * c + i * gg
        h2 = o * jnp.tanh(c2)
        out_ref[t, :] = h2[0]
        return (h2, c2)
    h0 = jnp.zeros((1, 192), jnp.float32)
    jax.lax.fori_loop(0, T_K, step, (h0, h0))


def _lstm_block(sub_f, pub_f, maccs_f, p0, p1, p2):
    ws = [p['Wih'].T for p in (p0, p1, p2)]
    bs = [p['bih'] + p['bhh'] for p in (p0, p1, p2)]
    bias = jnp.zeros((1, 768), jnp.float32)
    whh_bd = jnp.zeros((192, 768), jnp.float32)
    for l, p in enumerate((p0, p1, p2)):
        wt = p['Whh'].T
        for g in range(4):
            whh_bd = whh_bd.at[l * 64:(l + 1) * 64,
                               g * 192 + l * 64:g * 192 + (l + 1) * 64].set(
                                   wt[:, g * 64:(g + 1) * 64])
            bias = bias.at[0, g * 192 + l * 64:g * 192 + (l + 1) * 64].set(
                bs[l][g * 64:(g + 1) * 64])
    gin = pl.pallas_call(
        _lstm_proj_kernel,
        out_shape=jax.ShapeDtypeStruct((T_K, 768), jnp.float32),
    )(sub_f, pub_f, maccs_f, ws[0], ws[1], ws[2], bias)
    hs = pl.pallas_call(
        _lstm_rec_kernel,
        out_shape=jax.ShapeDtypeStruct((T_K, 192), jnp.float32),
    )(gin, whh_bd)
    return hs[:, 0:64], hs[:, 64:128], hs[:, 128:192]


def _bn_eval(x, p):
    return x / jnp.sqrt(1.0 + 1e-5) * p['gamma'] + p['beta']


def _fusion_kernel(fp_ref, pp_ref, gam_ref, bet_ref, wfp_ref, bfp_ref,
                   qr_ref, wg_ref, bg_ref, qs_ref, wf_ref, bf_ref,
                   ow_ref, ob_ref, out_ref):
    fp = fp_ref[...]
    acc = jnp.concatenate([pp_ref[0, :HALF_POOL], pp_ref[1, :HALF_POOL]],
                          axis=0)
    num = acc[:, 0:HID_K]
    cnt = acc[:, HID_K:HID_K + 1]
    g0 = num / jnp.maximum(cnt, 1.0)
    g = g0 / jnp.sqrt(1.0 + 1e-5) * gam_ref[...] + bet_ref[...]
    fpf = jnp.tanh(fp @ wfp_ref[...] + bfp_ref[...])
    grf = jnp.tanh(g @ wg_ref[...] + bg_ref[...])
    fa = jnp.exp(fpf @ qr_ref[...])
    ga = jnp.exp(grf @ qs_ref[...])
    den2 = fa + ga
    fa = fa / den2
    ga = ga / den2
    fx = jnp.concatenate(
        [fa[:, i:i + 1] * fpf + ga[:, i:i + 1] * grf
         for i in range(ATT_HEADS_K)], axis=1)
    fx = fx @ wf_ref[...] + bf_ref[...]
    out_ref[...] = fx @ ow_ref[...] + ob_ref[...]


def _pad1(a, n_pad, fill):
    return jnp.concatenate(
        [a.astype(jnp.int32), jnp.full((n_pad - a.shape[0],), fill, jnp.int32)])


def kernel(x, edge_attr, angle_attr, sub_f, pub_f, maccs_f, edge_index, angle_index, batch, params):
    src, dst = edge_index[0], edge_index[1]
    asrc, adst = angle_index[0], angle_index[1]

    src2 = _pad1(src, E_PAD, 0)
    dstg2 = _pad1(dst, E_PAD, 0)
    dsts2 = _split_idx(_pad1(dst, E_PAD, N_NODES_K), HALF_NODE, 5056)
    asrc2 = _pad1(asrc, A_PAD, 0)
    adstg2 = _pad1(adst, A_PAD, 0)
    ea_pad = jnp.zeros((E_PAD, 16), jnp.float32).at[:N_EDGES_K].set(edge_attr)
    aa_pad = jnp.zeros((A_PAD, 4), jnp.float32).at[:N_ANGLES_K].set(angle_attr)

    def node_conv(xlr, e_pad, p):
        gl, gr = _gather_pair(xlr, src2, dstg2, E_PAD)
        w = _edge_w(gl, gr, e_pad, p, E_PAD)
        return _sc_scatter_add(w, dsts2, NPH_NODE, E_PAD)

    def angle_conv(x_e_pad, p):
        xlr = _proj2_blocked(x_e_pad, p)
        gl, gr = _gather_pair(xlr, asrc2, adstg2, A_PAD)
        w = _edge_w(gl, gr, aa_pad, p, A_PAD)
        acc = jax.ops.segment_sum(w[:N_ANGLES_K, :HID_K + 1], adst,
                                  num_segments=N_EDGES_K)
        ba = acc[:, :HID_K] / (acc[:, HID_K:HID_K + 1] + 1e-16) + p['b']
        return jnp.zeros((E_PAD, HID_K), jnp.float32).at[:N_EDGES_K].set(ba)

    xlr1 = _proj2_small(x, params['conv1'])
    parts1 = node_conv(xlr1, ea_pad, params['conv1'])
    ba_pad = angle_conv(ea_pad, params['hconv1'])

    xlr2 = _finalize_proj(parts1, params['conv1']['b'], params['conv2'])
    parts2 = node_conv(xlr2, ba_pad, params['conv2'])
    ba_pad = angle_conv(ba_pad, params['hconv2'])

    xlr3 = _finalize_proj(parts2, params['conv2']['b'], params['conv3'])
    parts3 = node_conv(xlr3, ba_pad, params['conv3'])

    hrows = _finalize_pool(parts3, params['conv3']['b'])
    hrows_p = jnp.zeros((NPOOL, 128), jnp.float32).at[:NPAD_NODE].set(hrows)
    batch2 = _split_idx(_pad1(batch, NPOOL, NUM_GRAPHS_K), HALF_POOL,
                        HALF_POOL)
    pool_parts = _sc_scatter_add(hrows_p, batch2, NPH_POOL, NPOOL)

    s, pu, mc = _lstm_block(sub_f, pub_f, maccs_f,
                            params['lstm0'], params['lstm1'], params['lstm2'])
    fus = jnp.stack([s, pu, mc], axis=1)
    y = jnp.mean(fus, axis=(1, 2))
    yp = jnp.pad(y, 1)
    w = params['eca_w']
    cw = jax.nn.sigmoid(w[0] * yp[:-2] + w[1] * yp[1:-1] + w[2] * yp[2:])
    fp = cw[:, None] * jnp.sum(fus, axis=1)
    fp = _bn_eval(fp, params['f_bn'])

    a = params['att']
    out = pl.pallas_call(
        _fusion_kernel,
        out_shape=jax.ShapeDtypeStruct((NUM_GRAPHS_K, 17), jnp.float32),
    )(fp, pool_parts, params['g_bn']['gamma'][None, :],
      params['g_bn']['beta'][None, :],
      a['Wfp'], a['bfp'], a['qr'], a['Wg'], a['bg'], a['qs'],
      a['Wf'], a['bf'], params['out_W'], params['out_b'])
    return out
